# baseline (speedup 1.0000x reference)
import jax
import jax.numpy as jnp
import numpy as np
from jax import lax
from jax.experimental import pallas as pl
from jax.experimental.pallas import tpu as pltpu
from jax.experimental.pallas import tpu_sc as plsc

N = 1048576
K = 5
NW = 32
COLS_W = N // NW
CW = 2048
NCHUNK = COLS_W // CW

_LOG_COEF = [0.12644733488559723, -0.18256883323192596, 0.20221665501594543,
             -0.24957875907421112, 0.3333088159561157]
_MAGIC = 0x3F3504F3


def _fastlog_parts(x):
    b = plsc.bitcast(x, jnp.int32)
    e = (b - _MAGIC) >> 23
    m = plsc.bitcast(b - (e << 23), jnp.float32)
    ef = e.astype(jnp.float32)
    x1 = m - np.float32(1.0)
    z = x1 * x1
    p = np.float32(_LOG_COEF[0])
    for c in _LOG_COEF[1:]:
        p = p * x1 + np.float32(c)
    lnm = x1 + z * (x1 * p - np.float32(0.5))
    return lnm, ef


def _body(ut_hbm, ept_hbm, p16_hbm, ot_hbm,
          u0, u1, e0, e1, o0, o1, p16_v, sem0, sem1, oss0, oss1):
    wid = lax.axis_index("s") * 2 + lax.axis_index("c")
    base = wid * COLS_W

    pltpu.sync_copy(p16_hbm, p16_v.at[pl.ds(0, K)])
    iota = lax.iota(jnp.int32, 16)
    pv = p16_v[...]
    mx = jnp.max(jnp.where(iota < K, pv, np.float32(-3.0e38)))
    s = jnp.where(iota < K, jnp.exp(pv - mx), np.float32(0.0))
    se = jnp.zeros((16,), jnp.float32) + jnp.sum(s)
    lnm_se, ef_se = _fastlog_parts(se)
    lse = lnm_se + ef_se * np.float32(0.6931471805599453)
    tbv = jnp.where(iota < K, pv, np.float32(0.0)) - mx - lse \
        - np.float32(0.5 * K * np.log(2.0 * np.pi))
    nwv = -jnp.exp(-pv)
    nw2v = nwv * np.float32(0.6931471805599453)

    def splat(vec, k):
        return jnp.sum(jnp.where(iota == k, vec, np.float32(0.0)))

    nw = [splat(nwv, k) for k in range(K)]
    nw2 = [splat(nw2v, k) for k in range(K)]
    tb = [splat(tbv, k) for k in range(K)]
    muc = [jnp.full((16,), float(k), jnp.float32) for k in range(K)]

    ubuf, ebuf, sems = (u0, u1), (e0, e1), (sem0, sem1)
    obuf, osems = (o0, o1), (oss0, oss1)

    def start_in(c, b):
        col0 = base + c * CW
        pltpu.async_copy(ut_hbm.at[:, pl.ds(col0, CW)], ubuf[b], sems[b])
        pltpu.async_copy(ept_hbm.at[:, pl.ds(col0, CW)], ebuf[b], sems[b])

    def wait_in(b):
        pltpu.make_async_copy(ut_hbm.at[:, pl.ds(0, CW)], ubuf[b], sems[b]).wait()
        pltpu.make_async_copy(ept_hbm.at[:, pl.ds(0, CW)], ebuf[b], sems[b]).wait()

    def start_out(c, b):
        pltpu.async_copy(obuf[b], ot_hbm.at[:, pl.ds(base + c * CW, CW)], osems[b])

    def wait_out(b):
        pltpu.make_async_copy(obuf[b], ot_hbm.at[:, pl.ds(0, CW)], osems[b]).wait()

    def compute_chunk(uv, ev, ov):
        @plsc.parallel_loop(0, CW, step=16, unroll=8)
        def _(o):
            v = []
            for k in range(K):
                lnm, ef = _fastlog_parts(uv[k, pl.ds(o, 16)])
                v.append(lnm * nw[k] + ef * nw2[k])
            t01 = v[1] < v[0]
            m01 = jnp.where(t01, v[1], v[0])
            f01 = jnp.where(t01, muc[1], muc[0])
            b01 = jnp.where(t01, tb[1], tb[0])
            t23 = v[3] < v[2]
            m23 = jnp.where(t23, v[3], v[2])
            f23 = jnp.where(t23, muc[3], muc[2])
            b23 = jnp.where(t23, tb[3], tb[2])
            t03 = m23 < m01
            m03 = jnp.where(t03, m23, m01)
            f03 = jnp.where(t03, f23, f01)
            b03 = jnp.where(t03, b23, b01)
            t4 = v[4] < m03
            muf = jnp.where(t4, muc[4], f03)
            tbs = jnp.where(t4, tb[4], b03)

            ss = None
            for k in range(K):
                ek = ev[k, pl.ds(o, 16)]
                ov[k, pl.ds(o, 16)] = muf + ek
                ss = ek * ek if ss is None else ss + ek * ek
            ov[K, pl.ds(o, 16)] = tbs - np.float32(0.5) * ss

    start_in(0, 0)
    start_in(1, 1)

    def pair_body(i, _):
        c = i * 2

        wait_in(0)
        @pl.when(i > 0)
        def _():
            wait_out(0)
        compute_chunk(ubuf[0], ebuf[0], obuf[0])
        start_out(c, 0)
        start_in(c + 2, 0)

        wait_in(1)
        @pl.when(i > 0)
        def _():
            wait_out(1)
        compute_chunk(ubuf[1], ebuf[1], obuf[1])
        start_out(c + 1, 1)
        start_in(c + 3, 1)
        return 0

    lax.fori_loop(0, NCHUNK // 2 - 1, pair_body, 0)

    wait_in(0)
    wait_out(0)
    compute_chunk(ubuf[0], ebuf[0], obuf[0])
    start_out(NCHUNK - 2, 0)
    wait_in(1)
    wait_out(1)
    compute_chunk(ubuf[1], ebuf[1], obuf[1])
    start_out(NCHUNK - 1, 1)
    wait_out(0)
    wait_out(1)


@jax.jit
def _run(ut, ept, p16):
    mesh = plsc.VectorSubcoreMesh(
        core_axis_name="c", subcore_axis_name="s", num_cores=2, num_subcores=16)
    f = pl.kernel(
        _body,
        out_type=jax.ShapeDtypeStruct((K + 1, N), jnp.float32),
        mesh=mesh,
        scratch_types=[
            pltpu.VMEM((K, CW), jnp.float32),
            pltpu.VMEM((K, CW), jnp.float32),
            pltpu.VMEM((K, CW), jnp.float32),
            pltpu.VMEM((K, CW), jnp.float32),
            pltpu.VMEM((K + 1, CW), jnp.float32),
            pltpu.VMEM((K + 1, CW), jnp.float32),
            pltpu.VMEM((16,), jnp.float32),
            pltpu.SemaphoreType.DMA,
            pltpu.SemaphoreType.DMA,
            pltpu.SemaphoreType.DMA,
            pltpu.SemaphoreType.DMA,
        ],
        compiler_params=pltpu.CompilerParams(
            needs_layout_passes=False, use_tc_tiling_on_sc=True,
            disable_bounds_checks=True, skip_device_barrier=True),
    )
    return f(ut, ept, p16)


def kernel(u, eps, prob):
    return _run(u.T, eps.T, prob).T


# speedup vs baseline: 1.0009x; 1.0009x over previous
import jax
import jax.numpy as jnp
import numpy as np
from jax import lax
from jax.experimental import pallas as pl
from jax.experimental.pallas import tpu as pltpu
from jax.experimental.pallas import tpu_sc as plsc

N = 1048576
K = 5
NW = 32
COLS_W = N // NW
CW = 2048
NCHUNK = COLS_W // CW

_LOG_COEF = [0.12644733488559723, -0.18256883323192596, 0.20221665501594543,
             -0.24957875907421112, 0.3333088159561157]
_MAGIC = 0x3F3504F3


def _fastlog_parts(x):
    b = plsc.bitcast(x, jnp.int32)
    e = (b - _MAGIC) >> 23
    m = plsc.bitcast(b - (e << 23), jnp.float32)
    ef = e.astype(jnp.float32)
    x1 = m - np.float32(1.0)
    z = x1 * x1
    p = np.float32(_LOG_COEF[0])
    for c in _LOG_COEF[1:]:
        p = p * x1 + np.float32(c)
    lnm = x1 + z * (x1 * p - np.float32(0.5))
    return lnm, ef


def _body(ut_hbm, ept_hbm, p16_hbm, ot_hbm,
          u0, u1, e0, e1, o0, o1, p16_v, sem0, sem1, oss0, oss1):
    wid = lax.axis_index("s") * 2 + lax.axis_index("c")
    base = wid * COLS_W

    pltpu.sync_copy(p16_hbm, p16_v.at[pl.ds(0, K)])
    iota = lax.iota(jnp.int32, 16)
    pv = p16_v[...]
    mx = jnp.max(jnp.where(iota < K, pv, np.float32(-3.0e38)))
    s = jnp.where(iota < K, jnp.exp(pv - mx), np.float32(0.0))
    se = jnp.zeros((16,), jnp.float32) + jnp.sum(s)
    lnm_se, ef_se = _fastlog_parts(se)
    lse = lnm_se + ef_se * np.float32(0.6931471805599453)
    tbv = jnp.where(iota < K, pv, np.float32(0.0)) - mx - lse \
        - np.float32(0.5 * K * np.log(2.0 * np.pi))
    nwv = -jnp.exp(-pv)
    nw2v = nwv * np.float32(0.6931471805599453)

    def splat(vec, k):
        return jnp.sum(jnp.where(iota == k, vec, np.float32(0.0)))

    nw = [splat(nwv, k) for k in range(K)]
    nw2 = [splat(nw2v, k) for k in range(K)]
    tb = [splat(tbv, k) for k in range(K)]
    muc = [jnp.full((16,), float(k), jnp.float32) for k in range(K)]

    ubuf, ebuf, sems = (u0, u1), (e0, e1), (sem0, sem1)
    obuf, osems = (o0, o1), (oss0, oss1)

    def start_in(c, b):
        col0 = base + c * CW
        pltpu.async_copy(ut_hbm.at[:, pl.ds(col0, CW)], ubuf[b], sems[b])
        pltpu.async_copy(ept_hbm.at[:, pl.ds(col0, CW)], ebuf[b], sems[b])

    def wait_in(b):
        pltpu.make_async_copy(ut_hbm.at[:, pl.ds(0, CW)], ubuf[b], sems[b]).wait()
        pltpu.make_async_copy(ept_hbm.at[:, pl.ds(0, CW)], ebuf[b], sems[b]).wait()

    def start_out(c, b):
        pltpu.async_copy(obuf[b], ot_hbm.at[:, pl.ds(base + c * CW, CW)], osems[b])

    def wait_out(b):
        pltpu.make_async_copy(obuf[b], ot_hbm.at[:, pl.ds(0, CW)], osems[b]).wait()

    def compute_chunk(uv, ev, ov):
        @plsc.parallel_loop(0, CW, step=16, unroll=8)
        def _(o):
            v = []
            for k in range(K):
                lnm, ef = _fastlog_parts(uv[k, pl.ds(o, 16)])
                v.append(lnm * nw[k] + ef * nw2[k])
            t01 = v[1] < v[0]
            m01 = jnp.where(t01, v[1], v[0])
            f01 = jnp.where(t01, muc[1], muc[0])
            b01 = jnp.where(t01, tb[1], tb[0])
            t23 = v[3] < v[2]
            m23 = jnp.where(t23, v[3], v[2])
            f23 = jnp.where(t23, muc[3], muc[2])
            b23 = jnp.where(t23, tb[3], tb[2])
            t03 = m23 < m01
            m03 = jnp.where(t03, m23, m01)
            f03 = jnp.where(t03, f23, f01)
            b03 = jnp.where(t03, b23, b01)
            t4 = v[4] < m03
            muf = jnp.where(t4, muc[4], f03)
            tbs = jnp.where(t4, tb[4], b03)

            ss = None
            for k in range(K):
                ek = ev[k, pl.ds(o, 16)]
                ov[k, pl.ds(o, 16)] = muf + ek
                ss = ek * ek if ss is None else ss + ek * ek
            ov[K, pl.ds(o, 16)] = tbs - np.float32(0.5) * ss

    start_in(0, 0)
    start_in(1, 1)

    def pair_body(i, _):
        c = i * 2

        wait_in(0)
        @pl.when(i > 0)
        def _():
            wait_out(0)
        compute_chunk(ubuf[0], ebuf[0], obuf[0])
        start_out(c, 0)
        start_in(c + 2, 0)

        wait_in(1)
        @pl.when(i > 0)
        def _():
            wait_out(1)
        compute_chunk(ubuf[1], ebuf[1], obuf[1])
        start_out(c + 1, 1)
        start_in(c + 3, 1)
        return 0

    lax.fori_loop(0, NCHUNK // 2 - 1, pair_body, 0)

    wait_in(0)
    wait_out(0)
    compute_chunk(ubuf[0], ebuf[0], obuf[0])
    start_out(NCHUNK - 2, 0)
    wait_in(1)
    wait_out(1)
    compute_chunk(ubuf[1], ebuf[1], obuf[1])
    start_out(NCHUNK - 1, 1)
    wait_out(0)
    wait_out(1)


@jax.jit
def _run(ut, ept, p16):
    mesh = plsc.VectorSubcoreMesh(
        core_axis_name="c", subcore_axis_name="s", num_cores=2, num_subcores=16)
    f = pl.kernel(
        _body,
        out_type=jax.ShapeDtypeStruct((K + 1, N), jnp.float32),
        mesh=mesh,
        scratch_types=[
            pltpu.VMEM((K, CW), jnp.float32),
            pltpu.VMEM((K, CW), jnp.float32),
            pltpu.VMEM((K, CW), jnp.float32),
            pltpu.VMEM((K, CW), jnp.float32),
            pltpu.VMEM((K + 1, CW), jnp.float32),
            pltpu.VMEM((K + 1, CW), jnp.float32),
            pltpu.VMEM((16,), jnp.float32),
            pltpu.SemaphoreType.DMA,
            pltpu.SemaphoreType.DMA,
            pltpu.SemaphoreType.DMA,
            pltpu.SemaphoreType.DMA,
        ],
        compiler_params=pltpu.CompilerParams(
            needs_layout_passes=False, use_tc_tiling_on_sc=True),
    )
    return f(ut, ept, p16)


def kernel(u, eps, prob):
    return _run(u.T, eps.T, prob).T
